# baseline (speedup 1.0000x reference)
import functools

import jax
import jax.numpy as jnp
from jax import lax
from jax.experimental import pallas as pl
from jax.experimental.pallas import tpu as pltpu
from jax.experimental.pallas import tpu_sc as plsc

N_NODES = 10000
N_EDGES = 320000
D_IN = 128

NC = 2
NS = 16
L = 16

DEG_ROWS = 80
EDGE_COLS = 80
E_ROWS = N_EDGES // EDGE_COLS
P1_ROWS = 25
P1_STEPS = E_ROWS // NS // P1_ROWS
ROWS_PER_W = E_ROWS // (NC * NS)
NODES_PER_TILE = N_NODES // NS

_BC_DNUMS = lax.GatherDimensionNumbers(
    offset_dims=(), collapsed_slice_dims=(0,), start_index_map=(0,))


def _rsqrt16(x):
  i = lax.bitcast_convert_type(x, jnp.int32)
  i = jnp.int32(0x5F3759DF) - (i >> 1)
  y = lax.bitcast_convert_type(i, jnp.float32)
  for _ in range(3):
    y = y * (1.5 - 0.5 * x * y * y)
  return y


def _sc_aggregate(x, ei3d, ew2d):
  mesh = plsc.VectorSubcoreMesh(core_axis_name="c", subcore_axis_name="s")

  @functools.partial(
      pl.kernel,
      mesh=mesh,
      compiler_params=pltpu.CompilerParams(
          use_tc_tiling_on_sc=False, needs_layout_passes=False),
      out_type=[
          jax.ShapeDtypeStruct((N_NODES, D_IN), jnp.float32),
          jax.ShapeDtypeStruct((N_NODES, D_IN), jnp.float32),
      ],
      scratch_types=[
          pltpu.VMEM((DEG_ROWS, D_IN), jnp.float32),
          pltpu.VMEM((1, DEG_ROWS), jnp.int32),
          pltpu.VMEM((2, P1_ROWS, EDGE_COLS), jnp.int32),
          pltpu.VMEM((2, P1_ROWS, EDGE_COLS), jnp.float32),
          pltpu.VMEM((2, 1, EDGE_COLS), jnp.int32),
          pltpu.VMEM((2, 1, EDGE_COLS), jnp.int32),
          pltpu.VMEM((2, 1, EDGE_COLS), jnp.float32),
          pltpu.VMEM((2, 48), jnp.int32),
          pltpu.VMEM((2, 48), jnp.int32),
          pltpu.VMEM((96, D_IN), jnp.float32),
          pltpu.VMEM((96, D_IN), jnp.float32),
          pltpu.VMEM_SHARED((N_NODES, D_IN), jnp.float32),
          pltpu.SemaphoreType.DMA,
          pltpu.SemaphoreType.DMA,
          pltpu.SemaphoreType.DMA,
          pltpu.SemaphoreType.DMA,
          pltpu.SemaphoreType.DMA,
          pltpu.SemaphoreType.DMA,
          pltpu.SemaphoreType.DMA,
          pltpu.SemaphoreType.DMA,
      ],
  )
  def body(x_hbm, ei_hbm, ew_hbm, part0_hbm, part1_hbm,
           deg2d, idxbuf, cbuf, wbuf, epr, epc, epw,
           scidx0, scidx1, rows0, rows1, acc,
           semsc0, semsc1, semg0, semg1, seme0, seme1, semp0, semp1):
    c = lax.axis_index("c")
    s = lax.axis_index("s")
    wid = s * NC + c

    zf = jnp.zeros((L,), jnp.float32)
    iota = lax.iota(jnp.int32, L)


    @pl.loop(0, DEG_ROWS)
    def _(k):
      for f in range(D_IN // L):
        deg2d[k, pl.ds(L * f, L)] = zf

    @pl.loop(0, 96)
    def _(k):
      for f in range(D_IN // L):
        rows0[k, pl.ds(L * f, L)] = zf

    @pl.loop(EDGE_COLS, 96)
    def _(k):
      for f in range(D_IN // L):
        rows1[k, pl.ds(L * f, L)] = zf

    zi = jnp.zeros((L,), jnp.int32)
    scidx0[1, pl.ds(32, L)] = zi
    scidx1[1, pl.ds(32, L)] = zi

    for j in range(7):
      pltpu.sync_copy(
          rows0.at[pl.ds(0, EDGE_COLS)],
          acc.at[pl.ds(s * NODES_PER_TILE + j * EDGE_COLS, EDGE_COLS)])
    pltpu.sync_copy(
        rows0.at[pl.ds(0, NODES_PER_TILE - 7 * EDGE_COLS)],
        acc.at[pl.ds(s * NODES_PER_TILE + 7 * EDGE_COLS,
                     NODES_PER_TILE - 7 * EDGE_COLS)])

    for v in range(DEG_ROWS // L):
      idxbuf[0, pl.ds(L * v, L)] = iota + L * v

    p1_base = s * (P1_STEPS * P1_ROWS)

    def p1_issue(t, b, sem):
      pltpu.async_copy(ei_hbm.at[1, pl.ds(p1_base + t * P1_ROWS, P1_ROWS)],
                       cbuf.at[b], sem)
      pltpu.async_copy(ew_hbm.at[pl.ds(p1_base + t * P1_ROWS, P1_ROWS)],
                       wbuf.at[b], sem)

    def p1_wait(b, sem):
      pltpu.make_async_copy(ei_hbm.at[1, pl.ds(0, P1_ROWS)], cbuf.at[b],
                            sem).wait()
      pltpu.make_async_copy(ew_hbm.at[pl.ds(0, P1_ROWS)], wbuf.at[b],
                            sem).wait()

    def p1_compute(b):
      @pl.loop(0, P1_ROWS)
      def _(r):
        for g in range(EDGE_COLS // L):
          cv = cbuf[b, r, pl.ds(g * L, L)]
          wv = wbuf[b, r, pl.ds(g * L, L)]
          plsc.addupdate_scatter(deg2d, [cv >> 7, cv & 127], wv)

    p1_issue(0, 0, semp0)

    @pl.loop(0, P1_STEPS, step=2)
    def _(t):
      p1_wait(0, semp0)
      p1_issue(t + 1, 1, semp1)
      p1_compute(0)
      p1_wait(1, semp1)

      @pl.when(t + 2 <= P1_STEPS - 1)
      def _():
        p1_issue(t + 2, 0, semp0)

      p1_compute(1)

    plsc.subcore_barrier()

    pltpu.sync_copy(deg2d, acc.at[idxbuf.at[0]], add=True)

    plsc.subcore_barrier()

    pltpu.sync_copy(acc.at[pl.ds(0, DEG_ROWS)], deg2d)

    @pl.loop(0, DEG_ROWS)
    def _(k):
      for f in range(D_IN // L):
        deg2d[k, pl.ds(L * f, L)] = _rsqrt16(deg2d[k, pl.ds(L * f, L)] + 1.0)

    plsc.subcore_barrier()

    @pl.when(s == 0)
    def _():
      pltpu.sync_copy(rows0.at[pl.ds(0, DEG_ROWS)], acc.at[pl.ds(0, DEG_ROWS)])

    def store_sidx(scidx, g, vec):
      if g < 3:
        scidx[0, pl.ds(g * L, L)] = vec
      else:
        scidx[1, pl.ds((g - 3) * L, L)] = vec

    def scatter_chunk(scidx, rows):
      pltpu.sync_copy(rows.at[pl.ds(0, 48)], acc.at[scidx.at[0]], add=True)
      pltpu.sync_copy(rows.at[pl.ds(48, 48)], acc.at[scidx.at[1]], add=True)

    def scale_rows(nms, rows):
      for g in range(EDGE_COLS // L):
        nm = nms[g]
        for i in range(L):
          b = lax.gather(nm, jnp.full((L, 1), i, jnp.int32), _BC_DNUMS,
                         slice_sizes=(1,),
                         mode=lax.GatherScatterMode.PROMISE_IN_BOUNDS)
          e = g * L + i
          for f in range(D_IN // L):
            rows[e, pl.ds(L * f, L)] = rows[e, pl.ds(L * f, L)] * b

    plsc.subcore_barrier()

    sl_base = s * NODES_PER_TILE

    @pl.loop(c * 4, c * 4 + 4)
    def _(j):
      nb = sl_base + jnp.where(j < 7, j * EDGE_COLS, 545)
      pltpu.sync_copy(x_hbm.at[pl.ds(nb, EDGE_COLS)],
                      rows0.at[pl.ds(0, EDGE_COLS)])
      nms = []
      for v in range(EDGE_COLS // L):
        rv = iota + (nb + v * L)
        dv = plsc.load_gather(deg2d, [rv >> 7, rv & 127])
        nm = dv * dv
        if v == 0:
          nm = jnp.where(jnp.logical_and(j == 7, iota < 15), 0.0, nm)
        nms.append(nm)
        store_sidx(scidx0, v, rv)
      scale_rows(nms, rows0)
      scatter_chunk(scidx0, rows0)

    base = wid * ROWS_PER_W

    def issue_ep(n, b, sem):
      pltpu.async_copy(ei_hbm.at[0, pl.ds(base + n, 1)], epr.at[b], sem)
      pltpu.async_copy(ei_hbm.at[1, pl.ds(base + n, 1)], epc.at[b], sem)
      pltpu.async_copy(ew_hbm.at[pl.ds(base + n, 1)], epw.at[b], sem)

    def wait_ep(b, sem):
      pltpu.make_async_copy(ei_hbm.at[0, pl.ds(0, 1)], epr.at[b], sem).wait()
      pltpu.make_async_copy(ei_hbm.at[1, pl.ds(0, 1)], epc.at[b], sem).wait()
      pltpu.make_async_copy(ew_hbm.at[pl.ds(0, 1)], epw.at[b], sem).wait()

    def compute_norms(b, scidx):
      nms = []
      for g in range(EDGE_COLS // L):
        rv = epr[b, 0, pl.ds(g * L, L)]
        cv = epc[b, 0, pl.ds(g * L, L)]
        wv = epw[b, 0, pl.ds(g * L, L)]
        dr = plsc.load_gather(deg2d, [rv >> 7, rv & 127])
        dc = plsc.load_gather(deg2d, [cv >> 7, cv & 127])
        nms.append(dr * wv * dc)
        store_sidx(scidx, g, cv)
      return nms

    def issue_gather(b, rows, sem):
      pltpu.async_copy(x_hbm.at[epr.at[b, 0]], rows.at[pl.ds(0, EDGE_COLS)],
                       sem)

    def wait_gather(rows, sem):
      pltpu.make_async_copy(x_hbm.at[epr.at[0, 0]],
                            rows.at[pl.ds(0, EDGE_COLS)], sem).wait()

    def scale_scatter(nms, scidx, rows, sem):
      for g in range(3):
        nm = nms[g]
        for i in range(L):
          b = lax.gather(nm, jnp.full((L, 1), i, jnp.int32), _BC_DNUMS,
                         slice_sizes=(1,),
                         mode=lax.GatherScatterMode.PROMISE_IN_BOUNDS)
          e = g * L + i
          for f in range(D_IN // L):
            rows[e, pl.ds(L * f, L)] = rows[e, pl.ds(L * f, L)] * b
      pltpu.async_copy(rows.at[pl.ds(0, 48)], acc.at[scidx.at[0]], sem,
                       add=True)
      for g in range(3, EDGE_COLS // L):
        nm = nms[g]
        for i in range(L):
          b = lax.gather(nm, jnp.full((L, 1), i, jnp.int32), _BC_DNUMS,
                         slice_sizes=(1,),
                         mode=lax.GatherScatterMode.PROMISE_IN_BOUNDS)
          e = g * L + i
          for f in range(D_IN // L):
            rows[e, pl.ds(L * f, L)] = rows[e, pl.ds(L * f, L)] * b
      pltpu.async_copy(rows.at[pl.ds(48, 48)], acc.at[scidx.at[1]], sem,
                       add=True)
      pltpu.make_async_copy(rows.at[pl.ds(0, 48)], acc.at[scidx.at[0]],
                            sem).wait()
      pltpu.make_async_copy(rows.at[pl.ds(48, 48)], acc.at[scidx.at[1]],
                            sem).wait()

    pltpu.sync_copy(ei_hbm.at[0, pl.ds(base, 1)], epr.at[0])
    pltpu.sync_copy(ei_hbm.at[1, pl.ds(base, 1)], epc.at[0])
    pltpu.sync_copy(ew_hbm.at[pl.ds(base, 1)], epw.at[0])
    issue_gather(0, rows0, semg0)
    issue_ep(1, 1, seme1)

    @pl.loop(0, ROWS_PER_W - 1, step=2)
    def _(g):
      wait_ep(1, seme1)
      issue_gather(1, rows1, semg1)
      wait_gather(rows0, semg0)
      nms = compute_norms(0, scidx0)
      issue_ep(g + 2, 0, seme0)
      scale_scatter(nms, scidx0, rows0, semsc0)

      wait_ep(0, seme0)
      issue_gather(0, rows0, semg0)
      wait_gather(rows1, semg1)
      nms = compute_norms(1, scidx1)

      @pl.when(g + 3 <= ROWS_PER_W - 1)
      def _():
        issue_ep(g + 3, 1, seme1)

      scale_scatter(nms, scidx1, rows1, semsc1)

    wait_gather(rows0, semg0)
    nms = compute_norms(0, scidx0)
    scale_scatter(nms, scidx0, rows0, semsc0)

    plsc.subcore_barrier()

    @pl.when(c == 0)
    def _():
      pltpu.sync_copy(acc.at[pl.ds(s * NODES_PER_TILE, NODES_PER_TILE)],
                      part0_hbm.at[pl.ds(s * NODES_PER_TILE, NODES_PER_TILE)])

    @pl.when(c == 1)
    def _():
      pltpu.sync_copy(acc.at[pl.ds(s * NODES_PER_TILE, NODES_PER_TILE)],
                      part1_hbm.at[pl.ds(s * NODES_PER_TILE, NODES_PER_TILE)])

  return body(x, ei3d, ew2d)


def _tc_head_body(p0_ref, p1_ref, w1_ref, b1_ref, w2_ref, b2_ref, o_ref):
  agg = p0_ref[...] + p1_ref[...]
  h = jnp.dot(agg, w1_ref[...], preferred_element_type=jnp.float32)
  h = h + b1_ref[...]
  nrm = jnp.sqrt(jnp.sum(h * h, axis=1, keepdims=True))
  h = h / jnp.maximum(nrm, 1e-12)
  h = jnp.maximum(h, 0.0)
  o_ref[...] = (jnp.dot(h, w2_ref[...], preferred_element_type=jnp.float32)
                + b2_ref[...])


def _tc_head(p0, p1, W1, b1, W2, b2):
  blk = 2000
  grid = (N_NODES // blk,)
  n_cls = W2.shape[1]
  d_hid = W1.shape[1]
  return pl.pallas_call(
      _tc_head_body,
      grid=grid,
      in_specs=[
          pl.BlockSpec((blk, D_IN), lambda i: (i, 0)),
          pl.BlockSpec((blk, D_IN), lambda i: (i, 0)),
          pl.BlockSpec((D_IN, d_hid), lambda i: (0, 0)),
          pl.BlockSpec((1, d_hid), lambda i: (0, 0)),
          pl.BlockSpec((d_hid, n_cls), lambda i: (0, 0)),
          pl.BlockSpec((1, n_cls), lambda i: (0, 0)),
      ],
      out_specs=pl.BlockSpec((blk, n_cls), lambda i: (i, 0)),
      out_shape=jax.ShapeDtypeStruct((N_NODES, n_cls), jnp.float32),
  )(p0, p1, W1, b1.reshape(1, d_hid), W2, b2.reshape(1, n_cls))


def kernel(x, edge_index, edge_weights, W1, b1, W2, b2):
  ei3d = edge_index.astype(jnp.int32).reshape(2, E_ROWS, EDGE_COLS)
  ew2d = edge_weights.reshape(E_ROWS, EDGE_COLS)
  p0, p1 = _sc_aggregate(x, ei3d, ew2d)
  return _tc_head(p0, p1, W1, b1, W2, b2)


# speedup vs baseline: 1.1783x; 1.1783x over previous
import functools

import jax
import jax.numpy as jnp
from jax import lax
from jax.experimental import pallas as pl
from jax.experimental.pallas import tpu as pltpu
from jax.experimental.pallas import tpu_sc as plsc

N_NODES = 10000
N_EDGES = 320000
D_IN = 128

NC = 2
NS = 16
L = 16

DEG_ROWS = 80
EDGE_COLS = 80
E_ROWS = N_EDGES // EDGE_COLS
P1_ROWS = 25
P1_STEPS = E_ROWS // NS // P1_ROWS
ROWS_PER_W = E_ROWS // (NC * NS)
NODES_PER_TILE = N_NODES // NS

_BC_DNUMS = lax.GatherDimensionNumbers(
    offset_dims=(), collapsed_slice_dims=(0,), start_index_map=(0,))


def _rsqrt16(x):
  i = lax.bitcast_convert_type(x, jnp.int32)
  i = jnp.int32(0x5F3759DF) - (i >> 1)
  y = lax.bitcast_convert_type(i, jnp.float32)
  for _ in range(3):
    y = y * (1.5 - 0.5 * x * y * y)
  return y


def _sc_aggregate(x, ei3d, ew2d):
  mesh = plsc.VectorSubcoreMesh(core_axis_name="c", subcore_axis_name="s")

  @functools.partial(
      pl.kernel,
      mesh=mesh,
      compiler_params=pltpu.CompilerParams(
          use_tc_tiling_on_sc=False, needs_layout_passes=False),
      out_type=[
          jax.ShapeDtypeStruct((N_NODES, D_IN), jnp.float32),
          jax.ShapeDtypeStruct((N_NODES, D_IN), jnp.float32),
      ],
      scratch_types=[
          pltpu.VMEM((DEG_ROWS, D_IN), jnp.float32),
          pltpu.VMEM((DEG_ROWS, D_IN), jnp.float32),
          pltpu.VMEM((1, DEG_ROWS), jnp.int32),
          pltpu.VMEM((2, P1_ROWS, EDGE_COLS), jnp.int32),
          pltpu.VMEM((2, P1_ROWS, EDGE_COLS), jnp.float32),
          pltpu.VMEM((2, 1, EDGE_COLS), jnp.int32),
          pltpu.VMEM((2, 1, EDGE_COLS), jnp.int32),
          pltpu.VMEM((2, 1, EDGE_COLS), jnp.float32),
          pltpu.VMEM((1, EDGE_COLS), jnp.int32),
          pltpu.VMEM((1, EDGE_COLS), jnp.int32),
          pltpu.VMEM((EDGE_COLS, D_IN), jnp.float32),
          pltpu.VMEM((EDGE_COLS, D_IN), jnp.float32),
          pltpu.VMEM_SHARED((N_NODES, D_IN), jnp.float32),
          pltpu.SemaphoreType.DMA,
          pltpu.SemaphoreType.DMA,
          pltpu.SemaphoreType.DMA,
          pltpu.SemaphoreType.DMA,
          pltpu.SemaphoreType.DMA,
          pltpu.SemaphoreType.DMA,
      ],
  )
  def body(x_hbm, ei_hbm, ew_hbm, part0_hbm, part1_hbm,
           deg2d, dinv2d, idxbuf, cbuf, wbuf, epr, epc, epw,
           scidx0, scidx1, rows0, rows1, acc,
           semg0, semg1, seme0, seme1, semp0, semp1):
    c = lax.axis_index("c")
    s = lax.axis_index("s")
    wid = s * NC + c

    zf = jnp.zeros((L,), jnp.float32)
    iota = lax.iota(jnp.int32, L)


    @pl.loop(0, DEG_ROWS)
    def _(k):
      for f in range(D_IN // L):
        deg2d[k, pl.ds(L * f, L)] = zf

    @pl.loop(0, EDGE_COLS)
    def _(k):
      for f in range(D_IN // L):
        rows0[k, pl.ds(L * f, L)] = zf

    for j in range(7):
      pltpu.sync_copy(
          rows0, acc.at[pl.ds(s * NODES_PER_TILE + j * EDGE_COLS, EDGE_COLS)])
    pltpu.sync_copy(
        rows0.at[pl.ds(0, NODES_PER_TILE - 7 * EDGE_COLS)],
        acc.at[pl.ds(s * NODES_PER_TILE + 7 * EDGE_COLS,
                     NODES_PER_TILE - 7 * EDGE_COLS)])

    for v in range(DEG_ROWS // L):
      idxbuf[0, pl.ds(L * v, L)] = iota + L * v

    p1_base = s * (P1_STEPS * P1_ROWS)

    def p1_issue(t, b, sem):
      pltpu.async_copy(ei_hbm.at[1, pl.ds(p1_base + t * P1_ROWS, P1_ROWS)],
                       cbuf.at[b], sem)
      pltpu.async_copy(ew_hbm.at[pl.ds(p1_base + t * P1_ROWS, P1_ROWS)],
                       wbuf.at[b], sem)

    def p1_wait(b, sem):
      pltpu.make_async_copy(ei_hbm.at[1, pl.ds(0, P1_ROWS)], cbuf.at[b],
                            sem).wait()
      pltpu.make_async_copy(ew_hbm.at[pl.ds(0, P1_ROWS)], wbuf.at[b],
                            sem).wait()

    def p1_compute(b):
      @pl.loop(0, P1_ROWS)
      def _(r):
        for g in range(EDGE_COLS // L):
          cv = cbuf[b, r, pl.ds(g * L, L)]
          wv = wbuf[b, r, pl.ds(g * L, L)]
          plsc.addupdate_scatter(deg2d, [cv >> 7, cv & 127], wv)

    p1_issue(0, 0, semp0)

    @pl.loop(0, P1_STEPS, step=2)
    def _(t):
      p1_wait(0, semp0)
      p1_issue(t + 1, 1, semp1)
      p1_compute(0)
      p1_wait(1, semp1)

      @pl.when(t + 2 <= P1_STEPS - 1)
      def _():
        p1_issue(t + 2, 0, semp0)

      p1_compute(1)

    plsc.subcore_barrier()

    pltpu.sync_copy(deg2d, acc.at[idxbuf.at[0]], add=True)

    plsc.subcore_barrier()

    pltpu.sync_copy(acc.at[pl.ds(0, DEG_ROWS)], deg2d)

    @pl.loop(0, DEG_ROWS)
    def _(k):
      for f in range(D_IN // L):
        dinv2d[k, pl.ds(L * f, L)] = _rsqrt16(deg2d[k, pl.ds(L * f, L)] + 1.0)

    plsc.subcore_barrier()

    @pl.when(s == 0)
    def _():
      pltpu.sync_copy(rows0, acc.at[pl.ds(0, DEG_ROWS)])

    def scale_rows(nms, rows):
      for g in range(EDGE_COLS // L):
        nm = nms[g]
        for i in range(L):
          b = lax.gather(nm, jnp.full((L, 1), i, jnp.int32), _BC_DNUMS,
                         slice_sizes=(1,),
                         mode=lax.GatherScatterMode.PROMISE_IN_BOUNDS)
          e = g * L + i
          for f in range(D_IN // L):
            rows[e, pl.ds(L * f, L)] = rows[e, pl.ds(L * f, L)] * b

    plsc.subcore_barrier()

    sl_base = s * NODES_PER_TILE

    @pl.loop(c * 4, c * 4 + 4)
    def _(j):
      nb = sl_base + jnp.where(j < 7, j * EDGE_COLS, 545)
      pltpu.sync_copy(x_hbm.at[pl.ds(nb, EDGE_COLS)], rows0)
      nms = []
      for v in range(EDGE_COLS // L):
        rv = iota + (nb + v * L)
        dv = plsc.load_gather(dinv2d, [rv >> 7, rv & 127])
        nm = dv * dv
        if v == 0:
          nm = jnp.where(jnp.logical_and(j == 7, iota < 15), 0.0, nm)
        nms.append(nm)
        scidx0[0, pl.ds(v * L, L)] = rv
      scale_rows(nms, rows0)
      pltpu.sync_copy(rows0, acc.at[scidx0.at[0]], add=True)

    base = wid * ROWS_PER_W

    def issue_ep(n, b, sem):
      pltpu.async_copy(ei_hbm.at[0, pl.ds(base + n, 1)], epr.at[b], sem)
      pltpu.async_copy(ei_hbm.at[1, pl.ds(base + n, 1)], epc.at[b], sem)
      pltpu.async_copy(ew_hbm.at[pl.ds(base + n, 1)], epw.at[b], sem)

    def wait_ep(b, sem):
      pltpu.make_async_copy(ei_hbm.at[0, pl.ds(0, 1)], epr.at[b], sem).wait()
      pltpu.make_async_copy(ei_hbm.at[1, pl.ds(0, 1)], epc.at[b], sem).wait()
      pltpu.make_async_copy(ew_hbm.at[pl.ds(0, 1)], epw.at[b], sem).wait()

    def compute_norms(b, scidx):
      nms = []
      for g in range(EDGE_COLS // L):
        rv = epr[b, 0, pl.ds(g * L, L)]
        cv = epc[b, 0, pl.ds(g * L, L)]
        wv = epw[b, 0, pl.ds(g * L, L)]
        dr = plsc.load_gather(dinv2d, [rv >> 7, rv & 127])
        dc = plsc.load_gather(dinv2d, [cv >> 7, cv & 127])
        nms.append(dr * wv * dc)
        scidx[0, pl.ds(g * L, L)] = cv
      return nms

    def issue_gather(b, rows, sem):
      pltpu.async_copy(x_hbm.at[epr.at[b, 0]], rows, sem)

    def wait_gather(rows, sem):
      pltpu.make_async_copy(x_hbm.at[epr.at[0, 0]], rows, sem).wait()

    def scatter_chunk(scidx, rows):
      pltpu.sync_copy(rows, acc.at[scidx.at[0]], add=True)

    pltpu.sync_copy(ei_hbm.at[0, pl.ds(base, 1)], epr.at[0])
    pltpu.sync_copy(ei_hbm.at[1, pl.ds(base, 1)], epc.at[0])
    pltpu.sync_copy(ew_hbm.at[pl.ds(base, 1)], epw.at[0])
    issue_gather(0, rows0, semg0)
    issue_ep(1, 1, seme1)

    @pl.loop(0, ROWS_PER_W - 1, step=2)
    def _(g):
      wait_ep(1, seme1)
      issue_gather(1, rows1, semg1)
      wait_gather(rows0, semg0)
      nms = compute_norms(0, scidx0)
      issue_ep(g + 2, 0, seme0)
      scale_rows(nms, rows0)
      scatter_chunk(scidx0, rows0)

      wait_ep(0, seme0)
      issue_gather(0, rows0, semg0)
      wait_gather(rows1, semg1)
      nms = compute_norms(1, scidx1)

      @pl.when(g + 3 <= ROWS_PER_W - 1)
      def _():
        issue_ep(g + 3, 1, seme1)

      scale_rows(nms, rows1)
      scatter_chunk(scidx1, rows1)

    wait_gather(rows0, semg0)
    nms = compute_norms(0, scidx0)
    scale_rows(nms, rows0)
    scatter_chunk(scidx0, rows0)

    plsc.subcore_barrier()

    @pl.when(c == 0)
    def _():
      pltpu.sync_copy(acc.at[pl.ds(s * NODES_PER_TILE, NODES_PER_TILE)],
                      part0_hbm.at[pl.ds(s * NODES_PER_TILE, NODES_PER_TILE)])

    @pl.when(c == 1)
    def _():
      pltpu.sync_copy(acc.at[pl.ds(s * NODES_PER_TILE, NODES_PER_TILE)],
                      part1_hbm.at[pl.ds(s * NODES_PER_TILE, NODES_PER_TILE)])

  return body(x, ei3d, ew2d)


def _tc_head_body(p0_ref, p1_ref, w1_ref, b1_ref, w2_ref, b2_ref, o_ref):
  agg = p0_ref[...] + p1_ref[...]
  h = jnp.dot(agg, w1_ref[...], preferred_element_type=jnp.float32)
  h = h + b1_ref[...]
  nrm = jnp.sqrt(jnp.sum(h * h, axis=1, keepdims=True))
  h = h / jnp.maximum(nrm, 1e-12)
  h = jnp.maximum(h, 0.0)
  o_ref[...] = (jnp.dot(h, w2_ref[...], preferred_element_type=jnp.float32)
                + b2_ref[...])


def _tc_head(p0, p1, W1, b1, W2, b2):
  blk = 2000
  grid = (N_NODES // blk,)
  n_cls = W2.shape[1]
  d_hid = W1.shape[1]
  return pl.pallas_call(
      _tc_head_body,
      grid=grid,
      in_specs=[
          pl.BlockSpec((blk, D_IN), lambda i: (i, 0)),
          pl.BlockSpec((blk, D_IN), lambda i: (i, 0)),
          pl.BlockSpec((D_IN, d_hid), lambda i: (0, 0)),
          pl.BlockSpec((1, d_hid), lambda i: (0, 0)),
          pl.BlockSpec((d_hid, n_cls), lambda i: (0, 0)),
          pl.BlockSpec((1, n_cls), lambda i: (0, 0)),
      ],
      out_specs=pl.BlockSpec((blk, n_cls), lambda i: (i, 0)),
      out_shape=jax.ShapeDtypeStruct((N_NODES, n_cls), jnp.float32),
  )(p0, p1, W1, b1.reshape(1, d_hid), W2, b2.reshape(1, n_cls))


def kernel(x, edge_index, edge_weights, W1, b1, W2, b2):
  ei3d = edge_index.astype(jnp.int32).reshape(2, E_ROWS, EDGE_COLS)
  ew2d = edge_weights.reshape(E_ROWS, EDGE_COLS)
  p0, p1 = _sc_aggregate(x, ei3d, ew2d)
  return _tc_head(p0, p1, W1, b1, W2, b2)
